# baseline (speedup 1.0000x reference)
import functools

import jax
import jax.numpy as jnp
from jax import lax
from jax.experimental import pallas as pl
from jax.experimental.pallas import tpu as pltpu
from jax.experimental.pallas import tpu_sc as plsc

D_MODEL = 64
D_WIDE = 128
SCALE = 8.0
NUM_CORES = 2
NUM_SUBCORES = 16
LANES = 16
NUM_WORKERS = NUM_CORES * NUM_SUBCORES
NBUF = 2


def _embed_lookup(idx2, half, table_pairs, n_rows, n_cols):
    rows_per_w = n_rows // NUM_WORKERS
    n_tok = rows_per_w * n_cols
    assert n_rows % NUM_WORKERS == 0 and rows_per_w % NBUF == 0

    mesh = plsc.VectorSubcoreMesh(core_axis_name="c", subcore_axis_name="s")

    @functools.partial(
        pl.kernel,
        mesh=mesh,
        compiler_params=pltpu.CompilerParams(needs_layout_passes=False),
        out_type=jax.ShapeDtypeStruct((n_rows, n_cols, D_WIDE), jnp.float32),
        scratch_types=[
            pltpu.VMEM((n_tok,), jnp.int32),
            pltpu.VMEM((n_tok,), jnp.int32),
        ]
        + [pltpu.VMEM((n_cols, D_WIDE), jnp.float32)] * NBUF
        + [pltpu.SemaphoreType.DMA] * (2 * NBUF + 1),
    )
    def k(idx_hbm, half_hbm, table_hbm, out_hbm, idx_v, half_v, *bufs_and_sems):
        bufs = bufs_and_sems[:NBUF]
        gsem = bufs_and_sems[NBUF : 2 * NBUF]
        ssem = bufs_and_sems[2 * NBUF : 3 * NBUF]
        isem = bufs_and_sems[3 * NBUF]

        wid = lax.axis_index("s") * NUM_CORES + lax.axis_index("c")
        row0 = wid * rows_per_w
        for src, dst in ((idx_hbm, idx_v), (half_hbm, half_v)):
            pltpu.make_async_copy(
                src.at[pl.ds(row0 * n_cols, n_tok)], dst, isem
            ).start()
        for src, dst in ((idx_hbm, idx_v), (half_hbm, half_v)):
            pltpu.make_async_copy(
                src.at[pl.ds(row0 * n_cols, n_tok)], dst, isem
            ).wait()

        iota = lax.iota(jnp.int32, LANES)

        def gather_src(r):
            return table_hbm.at[idx_v.at[pl.ds(r * n_cols, n_cols)]]

        def out_dst(r):
            return out_hbm.at[row0 + r]

        for b in range(NBUF):
            pltpu.async_copy(gather_src(b), bufs[b], gsem[b])

        @pl.loop(0, rows_per_w, step=NBUF)
        def _(w):
            for b in range(NBUF):
                wb = w + b
                pltpu.make_async_copy(gather_src(wb), bufs[b], gsem[b]).wait()

                @pl.loop(0, n_cols)
                def _(r):
                    rv = r + iota * 0
                    pov = plsc.load_gather(half_v, [wb * n_cols + r + iota * 0])
                    for c in range(0, D_MODEL, LANES):
                        colv = pov + (c + iota)
                        v = plsc.load_gather(bufs[b], [rv, colv])
                        bufs[b].at[r, pl.ds(c, LANES)][...] = v * SCALE

                pltpu.async_copy(bufs[b], out_dst(wb), ssem[b])

            for b in range(NBUF):
                wb = w + b
                pltpu.make_async_copy(bufs[b], out_dst(wb), ssem[b]).wait()

                @pl.when(wb + NBUF < rows_per_w)
                def _():
                    pltpu.async_copy(gather_src(wb + NBUF), bufs[b], gsem[b])

    return k(idx2, half, table_pairs)


def kernel(x, table):
    n_rows, n_cols = x.shape
    xi = x.astype(jnp.int32)
    table_pairs = table.reshape(table.shape[0] // 2, D_WIDE)
    idx2 = (xi >> 1).reshape(n_rows * n_cols)
    half = ((xi & 1) << 6).reshape(n_rows * n_cols)
    out_wide = _embed_lookup(idx2, half, table_pairs, n_rows, n_cols)
    return out_wide[:, :, :D_MODEL]


# speedup vs baseline: 1.6309x; 1.6309x over previous
import functools

import jax
import jax.numpy as jnp
from jax import lax
from jax.experimental import pallas as pl
from jax.experimental.pallas import tpu as pltpu
from jax.experimental.pallas import tpu_sc as plsc

D_MODEL = 64
D_WIDE = 128
SCALE = 8.0
NUM_CORES = 2
NUM_SUBCORES = 16
LANES = 16
NUM_WORKERS = NUM_CORES * NUM_SUBCORES
NBUF = 4


def _embed_lookup(idx, table_wide, n_rows, n_cols):
    rows_per_w = n_rows // NUM_WORKERS
    n_tok = rows_per_w * n_cols
    assert n_rows % NUM_WORKERS == 0 and rows_per_w % NBUF == 0

    mesh = plsc.VectorSubcoreMesh(core_axis_name="c", subcore_axis_name="s")

    @functools.partial(
        pl.kernel,
        mesh=mesh,
        out_type=jax.ShapeDtypeStruct((n_rows, n_cols, D_WIDE), jnp.float32),
        scratch_types=[
            pltpu.VMEM((n_tok,), jnp.int32),
        ]
        + [pltpu.VMEM((n_cols, D_WIDE), jnp.float32)] * NBUF
        + [pltpu.SemaphoreType.DMA] * (2 * NBUF + 1),
    )
    def k(idx_hbm, table_hbm, out_hbm, idx_v, *bufs_and_sems):
        bufs = bufs_and_sems[:NBUF]
        gsem = bufs_and_sems[NBUF : 2 * NBUF]
        ssem = bufs_and_sems[2 * NBUF : 3 * NBUF]
        isem = bufs_and_sems[3 * NBUF]

        wid = lax.axis_index("s") * NUM_CORES + lax.axis_index("c")
        row0 = wid * rows_per_w
        pltpu.make_async_copy(
            idx_hbm.at[pl.ds(row0 * n_cols, n_tok)], idx_v, isem
        ).start()
        pltpu.make_async_copy(
            idx_hbm.at[pl.ds(row0 * n_cols, n_tok)], idx_v, isem
        ).wait()

        def gather_src(r):
            return table_hbm.at[idx_v.at[pl.ds(r * n_cols, n_cols)]]

        def out_dst(r):
            return out_hbm.at[row0 + r]

        for b in range(NBUF):
            pltpu.async_copy(gather_src(b), bufs[b], gsem[b])

        @pl.loop(0, rows_per_w, step=NBUF)
        def _(w):
            for b in range(NBUF):
                wb = w + b
                pltpu.make_async_copy(gather_src(wb), bufs[b], gsem[b]).wait()

                @pl.loop(0, n_cols)
                def _(r):
                    for c in range(0, D_MODEL, LANES):
                        slc = (pl.ds(r, 1), pl.ds(c, LANES))
                        bufs[b].at[slc][...] = bufs[b].at[slc][...] * SCALE

                pltpu.async_copy(bufs[b], out_dst(wb), ssem[b])

            for b in range(NBUF):
                wb = w + b
                pltpu.make_async_copy(bufs[b], out_dst(wb), ssem[b]).wait()

                @pl.when(wb + NBUF < rows_per_w)
                def _():
                    pltpu.async_copy(gather_src(wb + NBUF), bufs[b], gsem[b])

    return k(idx, table_wide)


def kernel(x, table):
    n_rows, n_cols = x.shape
    table_wide = jnp.pad(table, ((0, 0), (0, D_WIDE - D_MODEL)))
    idx = x.reshape(n_rows * n_cols).astype(jnp.int32)
    out_wide = _embed_lookup(idx, table_wide, n_rows, n_cols)
    return out_wide[:, :, :D_MODEL]
